# baseline (speedup 1.0000x reference)
import functools

import jax
import jax.numpy as jnp
from jax import lax
from jax.experimental import pallas as pl
from jax.experimental.pallas import tpu as pltpu
from jax.experimental.pallas import tpu_sc as plsc

N = 10000
E = 320000
NUM_GRAPHS = 64
D_IN = 3
H = 128
DENSE = 256

NC = 2
NS = 16
NW = NC * NS
NPAD = 10112
CHUNK = 128
CH0 = 103
CH1 = 55
CHM = max(CH0, CH1)
EP0 = NS * CH0 * CHUNK
EP1 = NS * CH1 * CHUNK
EP = EP0 + EP1
RPT = NPAD // NS

_HIGH = jax.lax.Precision.HIGHEST


def _mesh():
    return plsc.VectorSubcoreMesh(core_axis_name="c", subcore_axis_name="s")


def _core_stage(c, s, a0_hbm, a1_hbm, buf, sem):
    @pl.when(c == 0)
    def _():
        pltpu.async_copy(a0_hbm.at[s], buf.at[pl.ds(0, CH0)], sem)

    @pl.when(c != 0)
    def _():
        pltpu.async_copy(a1_hbm.at[s], buf.at[pl.ds(0, CH1)], sem)


def _core_stage_wait(c, s, a0_hbm, a1_hbm, buf, sem):
    @pl.when(c == 0)
    def _():
        pltpu.make_async_copy(a0_hbm.at[s], buf.at[pl.ds(0, CH0)], sem).wait()

    @pl.when(c != 0)
    def _():
        pltpu.make_async_copy(a1_hbm.at[s], buf.at[pl.ds(0, CH1)], sem).wait()


def _deg_kernel(ones_hbm, zeros_hbm, d0_hbm, d1_hbm, out_hbm,
                dst_v, ones_v, acc, sem_a, sem_b):
    c = lax.axis_index("c")
    s = lax.axis_index("s")
    _core_stage(c, s, d0_hbm, d1_hbm, dst_v, sem_a)
    cp_o = pltpu.async_copy(ones_hbm.at[pl.ds(0, CHUNK)], ones_v, sem_b)

    @pl.when(c == 0)
    def _():
        pltpu.sync_copy(ones_hbm.at[pl.ds(s * RPT, RPT)],
                        acc.at[pl.ds(s * RPT, RPT)])

    @pl.when(c != 0)
    def _():
        pltpu.sync_copy(zeros_hbm.at[pl.ds(s * RPT, RPT)],
                        acc.at[pl.ds(s * RPT, RPT)])

    _core_stage_wait(c, s, d0_hbm, d1_hbm, dst_v, sem_a)
    cp_o.wait()
    plsc.subcore_barrier()

    trip = jnp.where(c == 0, CH0, CH1)

    def body(j, carry):
        pltpu.sync_copy(ones_v, acc.at[dst_v.at[j]], add=True)
        return carry

    lax.fori_loop(0, trip, body, 0)
    plsc.subcore_barrier()
    pltpu.sync_copy(acc.at[pl.ds(s * RPT, RPT)],
                    out_hbm.at[c, pl.ds(s * RPT, RPT)])


def _deg(ones16, zeros16, d0, d1):
    k = functools.partial(
        pl.kernel,
        mesh=_mesh(),
        out_type=jax.ShapeDtypeStruct((NC, NPAD, 16), jnp.float32),
        scratch_types=[
            pltpu.VMEM((CHM, CHUNK), jnp.int32),
            pltpu.VMEM((CHUNK, 16), jnp.float32),
            pltpu.VMEM_SHARED((NPAD, 16), jnp.float32),
            pltpu.SemaphoreType.DMA,
            pltpu.SemaphoreType.DMA,
        ],
    )(_deg_kernel)
    return k(ones16, zeros16, d0, d1)


def _prop_kernel(D, g_hbm, zeros_hbm, s0_hbm, d0_hbm, s1_hbm, d1_hbm, out_hbm,
                 src_v, dst_v, rows_v, acc, sem_a, sem_b, sem_g):
    c = lax.axis_index("c")
    s = lax.axis_index("s")
    _core_stage(c, s, s0_hbm, s1_hbm, src_v, sem_a)
    _core_stage(c, s, d0_hbm, d1_hbm, dst_v, sem_b)

    @pl.when(c == 0)
    def _():
        pltpu.sync_copy(g_hbm.at[pl.ds(s * RPT, RPT)],
                        acc.at[pl.ds(s * RPT, RPT)])

    @pl.when(c != 0)
    def _():
        pltpu.sync_copy(zeros_hbm.at[pl.ds(s * RPT, RPT)],
                        acc.at[pl.ds(s * RPT, RPT)])

    _core_stage_wait(c, s, s0_hbm, s1_hbm, src_v, sem_a)
    _core_stage_wait(c, s, d0_hbm, d1_hbm, dst_v, sem_b)
    plsc.subcore_barrier()

    trip = jnp.where(c == 0, CH0, CH1)

    def body(j, carry):
        pltpu.async_copy(g_hbm.at[src_v.at[j]], rows_v, sem_g).wait()
        pltpu.sync_copy(rows_v, acc.at[dst_v.at[j]], add=True)
        return carry

    lax.fori_loop(0, trip, body, 0)
    plsc.subcore_barrier()
    pltpu.sync_copy(acc.at[pl.ds(s * RPT, RPT)],
                    out_hbm.at[c, pl.ds(s * RPT, RPT)])


def _prop(D, g, zerosD, s0, d0, s1, d1):
    k = functools.partial(
        pl.kernel,
        mesh=_mesh(),
        out_type=jax.ShapeDtypeStruct((NC, NPAD, D), jnp.float32),
        scratch_types=[
            pltpu.VMEM((CHM, CHUNK), jnp.int32),
            pltpu.VMEM((CHM, CHUNK), jnp.int32),
            pltpu.VMEM((CHUNK, D), jnp.float32),
            pltpu.VMEM_SHARED((NPAD, D), jnp.float32),
            pltpu.SemaphoreType.DMA,
            pltpu.SemaphoreType.DMA,
            pltpu.SemaphoreType.DMA,
        ],
    )(functools.partial(_prop_kernel, D))
    return k(g, zerosD, s0, d0, s1, d1)


def _k1_kernel(degp_ref, x_ref, dinv_ref, xs_ref):
    deg = degp_ref[0, :, 0:1] + degp_ref[1, :, 0:1]
    dinv = jax.lax.rsqrt(deg)
    dinv_ref[...] = dinv
    xs_ref[...] = dinv * x_ref[...]


def _k1(degp, x128):
    return pl.pallas_call(
        _k1_kernel,
        out_shape=(
            jax.ShapeDtypeStruct((NPAD, 1), jnp.float32),
            jax.ShapeDtypeStruct((NPAD, H), jnp.float32),
        ),
    )(degp, x128)


def _mm_kernel(s0_ref, s1_ref, dinv_ref, b_ref, Wa_ref, Wb_ref, out_ref):
    dinv = dinv_ref[...]
    y = dinv * (s0_ref[0] + s1_ref[0])
    if Wa_ref is not None:
        y = jnp.maximum(
            lax.dot(y, Wa_ref[...], precision=_HIGH,
                    preferred_element_type=jnp.float32) + b_ref[...], 0.0)
    else:
        y = jnp.maximum(y + b_ref[...], 0.0)
    out_ref[...] = dinv * lax.dot(y, Wb_ref[...], precision=_HIGH,
                                  preferred_element_type=jnp.float32)


def _mm2(sp, dinv, b0, W0p, W1):
    blk = 2528
    grid = (NPAD // blk,)
    return pl.pallas_call(
        _mm_kernel,
        grid=grid,
        in_specs=[
            pl.BlockSpec((1, blk, H), lambda i: (0, i, 0)),
            pl.BlockSpec((1, blk, H), lambda i: (1, i, 0)),
            pl.BlockSpec((blk, 1), lambda i: (i, 0)),
            pl.BlockSpec((H,), lambda i: (0,)),
            pl.BlockSpec((H, H), lambda i: (0, 0)),
            pl.BlockSpec((H, H), lambda i: (0, 0)),
        ],
        out_specs=pl.BlockSpec((blk, H), lambda i: (i, 0)),
        out_shape=jax.ShapeDtypeStruct((NPAD, H), jnp.float32),
    )(sp, sp, dinv, b0, W0p, W1)


def _mm128_kernel(s0_ref, s1_ref, dinv_ref, b_ref, Wb_ref, out_ref):
    _mm_kernel(s0_ref, s1_ref, dinv_ref, b_ref, None, Wb_ref, out_ref)


def _mm128(sp, dinv, b, W):
    blk = 2528
    grid = (NPAD // blk,)
    return pl.pallas_call(
        _mm128_kernel,
        grid=grid,
        in_specs=[
            pl.BlockSpec((1, blk, H), lambda i: (0, i, 0)),
            pl.BlockSpec((1, blk, H), lambda i: (1, i, 0)),
            pl.BlockSpec((blk, 1), lambda i: (i, 0)),
            pl.BlockSpec((H,), lambda i: (0,)),
            pl.BlockSpec((H, H), lambda i: (0, 0)),
        ],
        out_specs=pl.BlockSpec((blk, H), lambda i: (i, 0)),
        out_shape=jax.ShapeDtypeStruct((NPAD, H), jnp.float32),
    )(sp, sp, dinv, b, W)


def _head_kernel(s0_ref, s1_ref, dinv_ref, b2_ref, batch_ref,
                 fc1w_ref, fc1b_ref, fc2w_ref, fc2b_ref,
                 out_ref, sums_ref, counts_ref):
    i = pl.program_id(0)
    nblk = pl.num_programs(0)

    @pl.when(i == 0)
    def _init():
        sums_ref[...] = jnp.zeros_like(sums_ref)
        counts_ref[...] = jnp.zeros_like(counts_ref)

    h = jnp.maximum(dinv_ref[...] * (s0_ref[0] + s1_ref[0]) + b2_ref[...],
                    0.0)
    b = batch_ref[...]
    gid = jax.lax.broadcasted_iota(jnp.int32, (b.shape[0], NUM_GRAPHS), 1)
    m = (gid == b).astype(jnp.float32)
    dn = (((0,), (0,)), ((), ()))
    sums_ref[...] += lax.dot_general(m, h, dn, precision=_HIGH,
                                     preferred_element_type=jnp.float32)
    counts_ref[...] += lax.dot_general(
        m, jnp.ones((b.shape[0], 1), jnp.float32), dn, precision=_HIGH,
        preferred_element_type=jnp.float32)

    @pl.when(i == nblk - 1)
    def _fini():
        pooled = sums_ref[...] / jnp.maximum(counts_ref[...], 1.0)
        z = jnp.maximum(
            lax.dot(pooled, fc1w_ref[...], precision=_HIGH,
                    preferred_element_type=jnp.float32) + fc1b_ref[...], 0.0)
        out_ref[...] = (lax.dot(z, fc2w_ref[...], precision=_HIGH,
                                preferred_element_type=jnp.float32)
                        + fc2b_ref[...])


def _head(sp, dinv, b2, batch2, fc1_w, fc1_b, fc2_w, fc2_b):
    blk = 2528
    grid = (NPAD // blk,)
    return pl.pallas_call(
        _head_kernel,
        grid=grid,
        in_specs=[
            pl.BlockSpec((1, blk, H), lambda i: (0, i, 0)),
            pl.BlockSpec((1, blk, H), lambda i: (1, i, 0)),
            pl.BlockSpec((blk, 1), lambda i: (i, 0)),
            pl.BlockSpec((H,), lambda i: (0,)),
            pl.BlockSpec((blk, 1), lambda i: (i, 0)),
            pl.BlockSpec((H, DENSE), lambda i: (0, 0)),
            pl.BlockSpec((DENSE,), lambda i: (0,)),
            pl.BlockSpec((DENSE, 1), lambda i: (0, 0)),
            pl.BlockSpec((1,), lambda i: (0,)),
        ],
        out_specs=pl.BlockSpec((NUM_GRAPHS, 1), lambda i: (0, 0)),
        out_shape=jax.ShapeDtypeStruct((NUM_GRAPHS, 1), jnp.float32),
        scratch_shapes=[
            pltpu.VMEM((NUM_GRAPHS, H), jnp.float32),
            pltpu.VMEM((NUM_GRAPHS, 1), jnp.float32),
        ],
    )(sp, sp, dinv, b2, batch2, fc1_w, fc1_b, fc2_w, fc2_b)


def kernel(x, edge_index, batch, W0, b0, W1, b1, W2, b2,
           fc1_w, fc1_b, fc2_w, fc2_b):
    src = edge_index[0].astype(jnp.int32)
    dst = edge_index[1].astype(jnp.int32)
    pad = jnp.full((EP - E,), N, jnp.int32)
    srcp = jnp.concatenate([src, pad])
    dstp = jnp.concatenate([dst, pad])
    s0 = srcp[:EP0].reshape(NS, CH0, CHUNK)
    s1 = srcp[EP0:].reshape(NS, CH1, CHUNK)
    d0 = dstp[:EP0].reshape(NS, CH0, CHUNK)
    d1 = dstp[EP0:].reshape(NS, CH1, CHUNK)

    ones16 = jnp.ones((NPAD, 16), jnp.float32)
    zeros16 = jnp.zeros((NPAD, 16), jnp.float32)
    zeros128 = jnp.zeros((NPAD, H), jnp.float32)
    x128 = jnp.pad(x, ((0, NPAD - N), (0, H - D_IN)))
    W0p = jnp.pad(W0, ((0, H - D_IN), (0, 0)))
    batch2 = jnp.pad(batch.astype(jnp.int32), (0, NPAD - N),
                     constant_values=NUM_GRAPHS).reshape(NPAD, 1)

    degp = _deg(ones16, zeros16, d0, d1)
    dinv, xs = _k1(degp, x128)
    sp0 = _prop(H, xs, zeros128, s0, d0, s1, d1)
    g1 = _mm2(sp0, dinv, b0, W0p, W1)
    sp1 = _prop(H, g1, zeros128, s0, d0, s1, d1)
    g2 = _mm128(sp1, dinv, b1, W2)
    sp2 = _prop(H, g2, zeros128, s0, d0, s1, d1)
    return _head(sp2, dinv, b2, batch2, fc1_w, fc1_b, fc2_w, fc2_b)


# speedup vs baseline: 1.0563x; 1.0563x over previous
import functools

import jax
import jax.numpy as jnp
from jax import lax
from jax.experimental import pallas as pl
from jax.experimental.pallas import tpu as pltpu
from jax.experimental.pallas import tpu_sc as plsc

N = 10000
E = 320000
NUM_GRAPHS = 64
D_IN = 3
H = 128
DENSE = 256

NC = 2
NS = 16
NW = NC * NS
NPAD = 10112
CHUNK = 128
CH0 = 106
CH1 = 52
CHM = max(CH0, CH1)
EP0 = NS * CH0 * CHUNK
EP1 = NS * CH1 * CHUNK
EP = EP0 + EP1
RPT = NPAD // NS

_HIGH = jax.lax.Precision.HIGHEST


def _mesh():
    return plsc.VectorSubcoreMesh(core_axis_name="c", subcore_axis_name="s")


def _core_stage(c, s, a0_hbm, a1_hbm, buf, sem):
    @pl.when(c == 0)
    def _():
        pltpu.async_copy(a0_hbm.at[s], buf.at[pl.ds(0, CH0)], sem)

    @pl.when(c != 0)
    def _():
        pltpu.async_copy(a1_hbm.at[s], buf.at[pl.ds(0, CH1)], sem)


def _core_stage_wait(c, s, a0_hbm, a1_hbm, buf, sem):
    @pl.when(c == 0)
    def _():
        pltpu.make_async_copy(a0_hbm.at[s], buf.at[pl.ds(0, CH0)], sem).wait()

    @pl.when(c != 0)
    def _():
        pltpu.make_async_copy(a1_hbm.at[s], buf.at[pl.ds(0, CH1)], sem).wait()


def _deg_kernel(ones_hbm, zeros_hbm, d0_hbm, d1_hbm, out_hbm,
                dst_v, ones_v, acc, sem_a, sem_b):
    c = lax.axis_index("c")
    s = lax.axis_index("s")
    _core_stage(c, s, d0_hbm, d1_hbm, dst_v, sem_a)
    cp_o = pltpu.async_copy(ones_hbm.at[pl.ds(0, CHUNK)], ones_v, sem_b)

    @pl.when(c == 0)
    def _():
        pltpu.sync_copy(ones_hbm.at[pl.ds(s * RPT, RPT)],
                        acc.at[pl.ds(s * RPT, RPT)])

    @pl.when(c != 0)
    def _():
        pltpu.sync_copy(zeros_hbm.at[pl.ds(s * RPT, RPT)],
                        acc.at[pl.ds(s * RPT, RPT)])

    _core_stage_wait(c, s, d0_hbm, d1_hbm, dst_v, sem_a)
    cp_o.wait()
    plsc.subcore_barrier()

    trip = jnp.where(c == 0, CH0, CH1)

    def body(j, carry):
        pltpu.sync_copy(ones_v, acc.at[dst_v.at[j]], add=True)
        return carry

    lax.fori_loop(0, trip, body, 0)
    plsc.subcore_barrier()
    pltpu.sync_copy(acc.at[pl.ds(s * RPT, RPT)],
                    out_hbm.at[c, pl.ds(s * RPT, RPT)])


def _deg(ones16, zeros16, d0, d1):
    k = functools.partial(
        pl.kernel,
        mesh=_mesh(),
        out_type=jax.ShapeDtypeStruct((NC, NPAD, 16), jnp.float32),
        scratch_types=[
            pltpu.VMEM((CHM, CHUNK), jnp.int32),
            pltpu.VMEM((CHUNK, 16), jnp.float32),
            pltpu.VMEM_SHARED((NPAD, 16), jnp.float32),
            pltpu.SemaphoreType.DMA,
            pltpu.SemaphoreType.DMA,
        ],
    )(_deg_kernel)
    return k(ones16, zeros16, d0, d1)


def _prop_kernel(D, g_hbm, zeros_hbm, s0_hbm, d0_hbm, s1_hbm, d1_hbm, out_hbm,
                 src_v, dst_v, rows_v, acc, sem_a, sem_b, sem_g):
    c = lax.axis_index("c")
    s = lax.axis_index("s")
    _core_stage(c, s, s0_hbm, s1_hbm, src_v, sem_a)
    _core_stage(c, s, d0_hbm, d1_hbm, dst_v, sem_b)

    @pl.when(c == 0)
    def _():
        pltpu.sync_copy(g_hbm.at[pl.ds(s * RPT, RPT)],
                        acc.at[pl.ds(s * RPT, RPT)])

    @pl.when(c != 0)
    def _():
        pltpu.sync_copy(zeros_hbm.at[pl.ds(s * RPT, RPT)],
                        acc.at[pl.ds(s * RPT, RPT)])

    _core_stage_wait(c, s, s0_hbm, s1_hbm, src_v, sem_a)
    _core_stage_wait(c, s, d0_hbm, d1_hbm, dst_v, sem_b)
    plsc.subcore_barrier()

    trip = jnp.where(c == 0, CH0, CH1)

    def body(j, carry):
        pltpu.async_copy(g_hbm.at[src_v.at[j]], rows_v, sem_g).wait()
        pltpu.sync_copy(rows_v, acc.at[dst_v.at[j]], add=True)
        return carry

    lax.fori_loop(0, trip, body, 0)
    plsc.subcore_barrier()
    pltpu.sync_copy(acc.at[pl.ds(s * RPT, RPT)],
                    out_hbm.at[c, pl.ds(s * RPT, RPT)])


def _prop(D, g, zerosD, s0, d0, s1, d1):
    k = functools.partial(
        pl.kernel,
        mesh=_mesh(),
        out_type=jax.ShapeDtypeStruct((NC, NPAD, D), jnp.float32),
        scratch_types=[
            pltpu.VMEM((CHM, CHUNK), jnp.int32),
            pltpu.VMEM((CHM, CHUNK), jnp.int32),
            pltpu.VMEM((CHUNK, D), jnp.float32),
            pltpu.VMEM_SHARED((NPAD, D), jnp.float32),
            pltpu.SemaphoreType.DMA,
            pltpu.SemaphoreType.DMA,
            pltpu.SemaphoreType.DMA,
        ],
    )(functools.partial(_prop_kernel, D))
    return k(g, zerosD, s0, d0, s1, d1)


def _k1_kernel(degp_ref, x_ref, dinv_ref, xs_ref):
    deg = degp_ref[0, :, 0:1] + degp_ref[1, :, 0:1]
    dinv = jax.lax.rsqrt(deg)
    dinv_ref[...] = dinv
    xs_ref[...] = dinv * x_ref[...]


def _k1(degp, x128):
    return pl.pallas_call(
        _k1_kernel,
        out_shape=(
            jax.ShapeDtypeStruct((NPAD, 1), jnp.float32),
            jax.ShapeDtypeStruct((NPAD, H), jnp.float32),
        ),
    )(degp, x128)


def _mm_kernel(s0_ref, s1_ref, dinv_ref, b_ref, Wa_ref, Wb_ref, out_ref):
    dinv = dinv_ref[...]
    y = dinv * (s0_ref[0] + s1_ref[0])
    if Wa_ref is not None:
        y = jnp.maximum(
            lax.dot(y, Wa_ref[...], precision=_HIGH,
                    preferred_element_type=jnp.float32) + b_ref[...], 0.0)
    else:
        y = jnp.maximum(y + b_ref[...], 0.0)
    out_ref[...] = dinv * lax.dot(y, Wb_ref[...], precision=_HIGH,
                                  preferred_element_type=jnp.float32)


def _mm2(sp, dinv, b0, W0p, W1):
    blk = 2528
    grid = (NPAD // blk,)
    return pl.pallas_call(
        _mm_kernel,
        grid=grid,
        in_specs=[
            pl.BlockSpec((1, blk, H), lambda i: (0, i, 0)),
            pl.BlockSpec((1, blk, H), lambda i: (1, i, 0)),
            pl.BlockSpec((blk, 1), lambda i: (i, 0)),
            pl.BlockSpec((H,), lambda i: (0,)),
            pl.BlockSpec((H, H), lambda i: (0, 0)),
            pl.BlockSpec((H, H), lambda i: (0, 0)),
        ],
        out_specs=pl.BlockSpec((blk, H), lambda i: (i, 0)),
        out_shape=jax.ShapeDtypeStruct((NPAD, H), jnp.float32),
    )(sp, sp, dinv, b0, W0p, W1)


def _mm128_kernel(s0_ref, s1_ref, dinv_ref, b_ref, Wb_ref, out_ref):
    _mm_kernel(s0_ref, s1_ref, dinv_ref, b_ref, None, Wb_ref, out_ref)


def _mm128(sp, dinv, b, W):
    blk = 2528
    grid = (NPAD // blk,)
    return pl.pallas_call(
        _mm128_kernel,
        grid=grid,
        in_specs=[
            pl.BlockSpec((1, blk, H), lambda i: (0, i, 0)),
            pl.BlockSpec((1, blk, H), lambda i: (1, i, 0)),
            pl.BlockSpec((blk, 1), lambda i: (i, 0)),
            pl.BlockSpec((H,), lambda i: (0,)),
            pl.BlockSpec((H, H), lambda i: (0, 0)),
        ],
        out_specs=pl.BlockSpec((blk, H), lambda i: (i, 0)),
        out_shape=jax.ShapeDtypeStruct((NPAD, H), jnp.float32),
    )(sp, sp, dinv, b, W)


def _head_kernel(s0_ref, s1_ref, dinv_ref, b2_ref, batch_ref,
                 fc1w_ref, fc1b_ref, fc2w_ref, fc2b_ref,
                 out_ref, sums_ref, counts_ref):
    i = pl.program_id(0)
    nblk = pl.num_programs(0)

    @pl.when(i == 0)
    def _init():
        sums_ref[...] = jnp.zeros_like(sums_ref)
        counts_ref[...] = jnp.zeros_like(counts_ref)

    h = jnp.maximum(dinv_ref[...] * (s0_ref[0] + s1_ref[0]) + b2_ref[...],
                    0.0)
    b = batch_ref[...]
    gid = jax.lax.broadcasted_iota(jnp.int32, (b.shape[0], NUM_GRAPHS), 1)
    m = (gid == b).astype(jnp.float32)
    dn = (((0,), (0,)), ((), ()))
    sums_ref[...] += lax.dot_general(m, h, dn, precision=_HIGH,
                                     preferred_element_type=jnp.float32)
    counts_ref[...] += lax.dot_general(
        m, jnp.ones((b.shape[0], 1), jnp.float32), dn, precision=_HIGH,
        preferred_element_type=jnp.float32)

    @pl.when(i == nblk - 1)
    def _fini():
        pooled = sums_ref[...] / jnp.maximum(counts_ref[...], 1.0)
        z = jnp.maximum(
            lax.dot(pooled, fc1w_ref[...], precision=_HIGH,
                    preferred_element_type=jnp.float32) + fc1b_ref[...], 0.0)
        out_ref[...] = (lax.dot(z, fc2w_ref[...], precision=_HIGH,
                                preferred_element_type=jnp.float32)
                        + fc2b_ref[...])


def _head(sp, dinv, b2, batch2, fc1_w, fc1_b, fc2_w, fc2_b):
    blk = 2528
    grid = (NPAD // blk,)
    return pl.pallas_call(
        _head_kernel,
        grid=grid,
        in_specs=[
            pl.BlockSpec((1, blk, H), lambda i: (0, i, 0)),
            pl.BlockSpec((1, blk, H), lambda i: (1, i, 0)),
            pl.BlockSpec((blk, 1), lambda i: (i, 0)),
            pl.BlockSpec((H,), lambda i: (0,)),
            pl.BlockSpec((blk, 1), lambda i: (i, 0)),
            pl.BlockSpec((H, DENSE), lambda i: (0, 0)),
            pl.BlockSpec((DENSE,), lambda i: (0,)),
            pl.BlockSpec((DENSE, 1), lambda i: (0, 0)),
            pl.BlockSpec((1,), lambda i: (0,)),
        ],
        out_specs=pl.BlockSpec((NUM_GRAPHS, 1), lambda i: (0, 0)),
        out_shape=jax.ShapeDtypeStruct((NUM_GRAPHS, 1), jnp.float32),
        scratch_shapes=[
            pltpu.VMEM((NUM_GRAPHS, H), jnp.float32),
            pltpu.VMEM((NUM_GRAPHS, 1), jnp.float32),
        ],
    )(sp, sp, dinv, b2, batch2, fc1_w, fc1_b, fc2_w, fc2_b)


def kernel(x, edge_index, batch, W0, b0, W1, b1, W2, b2,
           fc1_w, fc1_b, fc2_w, fc2_b):
    src = edge_index[0].astype(jnp.int32)
    dst = edge_index[1].astype(jnp.int32)
    pad = jnp.full((EP - E,), N, jnp.int32)
    srcp = jnp.concatenate([src, pad])
    dstp = jnp.concatenate([dst, pad])
    s0 = srcp[:EP0].reshape(NS, CH0, CHUNK)
    s1 = srcp[EP0:].reshape(NS, CH1, CHUNK)
    d0 = dstp[:EP0].reshape(NS, CH0, CHUNK)
    d1 = dstp[EP0:].reshape(NS, CH1, CHUNK)

    ones16 = jnp.ones((NPAD, 16), jnp.float32)
    zeros16 = jnp.zeros((NPAD, 16), jnp.float32)
    zeros128 = jnp.zeros((NPAD, H), jnp.float32)
    x128 = jnp.pad(x, ((0, NPAD - N), (0, H - D_IN)))
    W0p = jnp.pad(W0, ((0, H - D_IN), (0, 0)))
    batch2 = jnp.pad(batch.astype(jnp.int32), (0, NPAD - N),
                     constant_values=NUM_GRAPHS).reshape(NPAD, 1)

    degp = _deg(ones16, zeros16, d0, d1)
    dinv, xs = _k1(degp, x128)
    sp0 = _prop(H, xs, zeros128, s0, d0, s1, d1)
    g1 = _mm2(sp0, dinv, b0, W0p, W1)
    sp1 = _prop(H, g1, zeros128, s0, d0, s1, d1)
    g2 = _mm128(sp1, dinv, b1, W2)
    sp2 = _prop(H, g2, zeros128, s0, d0, s1, d1)
    return _head(sp2, dinv, b2, batch2, fc1_w, fc1_b, fc2_w, fc2_b)
